# baseline (speedup 1.0000x reference)
import functools

import jax
import jax.numpy as jnp
from jax import lax
from jax.experimental import pallas as pl
from jax.experimental.pallas import tpu as pltpu
from jax.experimental.pallas import tpu_sc as plsc

_HIDDEN = 768
_NUM_EXPERTS = 8
_EXPANDED = 1536
_N_TOKENS = 128
_NC = 2
_L = 16
_CHUNKS = _N_TOKENS // _L


def _logits_body(tokens_ref, rwt_ref, rb_ref, out_ref):
    out_ref[...] = lax.dot_general(
        rwt_ref[...], tokens_ref[...],
        dimension_numbers=(((1,), (1,)), ((), ())),
        preferred_element_type=jnp.float32) + rb_ref[...][:, None]


def _compute_logits_t(tokens, router_w, router_b):
    return pl.pallas_call(
        _logits_body,
        in_specs=[
            pl.BlockSpec((_N_TOKENS, _HIDDEN), lambda: (0, 0)),
            pl.BlockSpec((_NUM_EXPERTS, _HIDDEN), lambda: (0, 0)),
            pl.BlockSpec((_NUM_EXPERTS,), lambda: (0,)),
        ],
        out_specs=pl.BlockSpec((_NUM_EXPERTS, _N_TOKENS), lambda: (0, 0)),
        out_shape=jax.ShapeDtypeStruct((_NUM_EXPERTS, _N_TOKENS),
                                       jnp.float32),
    )(tokens, router_w.T, router_b)


_sc_mesh = plsc.VectorSubcoreMesh(core_axis_name="c", subcore_axis_name="s")


@functools.partial(
    pl.kernel,
    mesh=_sc_mesh,
    out_type=jax.ShapeDtypeStruct((_NUM_EXPERTS, _N_TOKENS), jnp.float32),
    scratch_types=[
        pltpu.VMEM((_NUM_EXPERTS, _N_TOKENS), jnp.float32),
        pltpu.VMEM((_NUM_EXPERTS, _N_TOKENS), jnp.float32),
    ],
)
def _sc_router(logits_hbm, pt_hbm, lg_v, pt_v):
    wid = lax.axis_index("s") * _NC + lax.axis_index("c")

    @pl.when(wid == 0)
    def _():
        pltpu.sync_copy(logits_hbm, lg_v)
        zero = jnp.zeros((_L,), jnp.float32)

        def chunk(j, carry):
            sl = pl.ds(j * _L, _L)
            m1 = lg_v[0, sl]
            i1 = jnp.zeros((_L,), jnp.int32)
            for e in range(1, _NUM_EXPERTS):
                v = lg_v[e, sl]
                b = v > m1
                m1 = jnp.where(b, v, m1)
                i1 = jnp.where(b, e, i1)
            m2 = jnp.full((_L,), -jnp.inf, jnp.float32)
            i2 = jnp.zeros((_L,), jnp.int32)
            for e in range(_NUM_EXPERTS):
                v = lg_v[e, sl]
                b = jnp.logical_and(i1 != e, v > m2)
                m2 = jnp.where(b, v, m2)
                i2 = jnp.where(b, e, i2)
            p1 = 1.0 / (1.0 + jnp.exp(m2 - m1))
            p2 = 1.0 - p1
            for e in range(_NUM_EXPERTS):
                pt_v[e, sl] = jnp.where(i1 == e, p1,
                                        jnp.where(i2 == e, p2, zero))
            return carry

        lax.fori_loop(0, _CHUNKS, chunk, 0)
        pltpu.sync_copy(pt_v, pt_hbm)


def _row(ref, e, width):
    rows = lax.broadcasted_iota(jnp.int32, (_NUM_EXPERTS, width), 0)
    return jnp.sum(jnp.where(rows == e, ref[...], 0.0), axis=0,
                   keepdims=True)


_SPLIT = 4
_H_SP = _HIDDEN // _SPLIT
_E_SP = _EXPANDED // _SPLIT


def _ffn_all_body(tokens_ref, *refs):
    w1_refs = refs[:_SPLIT]
    b1_ref = refs[_SPLIT]
    w2_refs = refs[_SPLIT + 1:2 * _SPLIT + 1]
    b2_ref = refs[2 * _SPLIT + 1]
    o_ref = refs[2 * _SPLIT + 2]
    e = pl.program_id(0)
    tokens = tokens_ref[...]
    h = _row(b1_ref, e, _EXPANDED)
    for k in range(_SPLIT):
        h += jnp.dot(tokens[:, k * _H_SP:(k + 1) * _H_SP], w1_refs[k][0],
                     preferred_element_type=jnp.float32)
    h = h * 0.5 * (1.0 + lax.erf(h * 0.7071067811865476))
    o = _row(b2_ref, e, _HIDDEN)
    for k in range(_SPLIT):
        o += jnp.dot(h[:, k * _E_SP:(k + 1) * _E_SP], w2_refs[k][0],
                     preferred_element_type=jnp.float32)
    o_ref[0] = o


def _ffn_all(tokens, w1, b1, w2, b2):
    w1_specs = [
        pl.BlockSpec((1, _H_SP, _EXPANDED), lambda e, k=k: (e, k, 0))
        for k in range(_SPLIT)
    ]
    w2_specs = [
        pl.BlockSpec((1, _E_SP, _HIDDEN), lambda e, k=k: (e, k, 0))
        for k in range(_SPLIT)
    ]
    return pl.pallas_call(
        _ffn_all_body,
        grid=(_NUM_EXPERTS,),
        in_specs=[
            pl.BlockSpec((_N_TOKENS, _HIDDEN), lambda e: (0, 0)),
            *w1_specs,
            pl.BlockSpec((_NUM_EXPERTS, _EXPANDED), lambda e: (0, 0)),
            *w2_specs,
            pl.BlockSpec((_NUM_EXPERTS, _HIDDEN), lambda e: (0, 0)),
        ],
        out_specs=pl.BlockSpec((1, _N_TOKENS, _HIDDEN), lambda e: (e, 0, 0)),
        out_shape=jax.ShapeDtypeStruct((_NUM_EXPERTS, _N_TOKENS, _HIDDEN),
                                       tokens.dtype),
    )(tokens, *([w1] * _SPLIT), b1, *([w2] * _SPLIT), b2)


def _combine_body(o_ref, pt_ref, out_ref):
    p = jnp.transpose(pt_ref[...])
    acc = p[:, 0:1] * o_ref[0]
    for e in range(1, _NUM_EXPERTS):
        acc += p[:, e:e + 1] * o_ref[e]
    out_ref[...] = acc


def _combine(o_all, pt):
    return pl.pallas_call(
        _combine_body,
        in_specs=[
            pl.BlockSpec((_NUM_EXPERTS, _N_TOKENS, _HIDDEN),
                         lambda: (0, 0, 0)),
            pl.BlockSpec((_NUM_EXPERTS, _N_TOKENS), lambda: (0, 0)),
        ],
        out_specs=pl.BlockSpec((_N_TOKENS, _HIDDEN), lambda: (0, 0)),
        out_shape=jax.ShapeDtypeStruct((_N_TOKENS, _HIDDEN), o_all.dtype),
    )(o_all, pt)


def kernel(tokens, router_w, router_b, w1, b1, w2, b2):
    logits_t = _compute_logits_t(tokens, router_w, router_b)
    pt = _sc_router(logits_t)
    o_all = _ffn_all(tokens, w1, b1, w2, b2)
    return _combine(o_all, pt)


# speedup vs baseline: 1.0355x; 1.0355x over previous
import functools

import jax
import jax.numpy as jnp
from jax import lax
from jax.experimental import pallas as pl
from jax.experimental.pallas import tpu as pltpu
from jax.experimental.pallas import tpu_sc as plsc

_HIDDEN = 768
_NUM_EXPERTS = 8
_EXPANDED = 1536
_N_TOKENS = 128
_NC = 2
_L = 16
_CHUNKS = _N_TOKENS // _L


def _logits_body(tokens_ref, rwt_ref, rb_ref, out_ref):
    out_ref[...] = lax.dot_general(
        rwt_ref[...], tokens_ref[...],
        dimension_numbers=(((1,), (1,)), ((), ())),
        preferred_element_type=jnp.float32) + rb_ref[...][:, None]


def _compute_logits_t(tokens, router_w, router_b):
    return pl.pallas_call(
        _logits_body,
        in_specs=[
            pl.BlockSpec((_N_TOKENS, _HIDDEN), lambda: (0, 0)),
            pl.BlockSpec((_NUM_EXPERTS, _HIDDEN), lambda: (0, 0)),
            pl.BlockSpec((_NUM_EXPERTS,), lambda: (0,)),
        ],
        out_specs=pl.BlockSpec((_NUM_EXPERTS, _N_TOKENS), lambda: (0, 0)),
        out_shape=jax.ShapeDtypeStruct((_NUM_EXPERTS, _N_TOKENS),
                                       jnp.float32),
    )(tokens, router_w.T, router_b)


_sc_mesh = plsc.VectorSubcoreMesh(core_axis_name="c", subcore_axis_name="s")


@functools.partial(
    pl.kernel,
    mesh=_sc_mesh,
    out_type=jax.ShapeDtypeStruct((_NUM_EXPERTS, _N_TOKENS), jnp.float32),
    scratch_types=[
        pltpu.VMEM((_NUM_EXPERTS, _N_TOKENS), jnp.float32),
        pltpu.VMEM((_NUM_EXPERTS, _N_TOKENS), jnp.float32),
    ],
)
def _sc_router(logits_hbm, pt_hbm, lg_v, pt_v):
    wid = lax.axis_index("s") * _NC + lax.axis_index("c")

    @pl.when(wid == 0)
    def _():
        pltpu.sync_copy(logits_hbm, lg_v)
        zero = jnp.zeros((_L,), jnp.float32)

        def chunk(j, carry):
            sl = pl.ds(j * _L, _L)
            m1 = lg_v[0, sl]
            i1 = jnp.zeros((_L,), jnp.int32)
            for e in range(1, _NUM_EXPERTS):
                v = lg_v[e, sl]
                b = v > m1
                m1 = jnp.where(b, v, m1)
                i1 = jnp.where(b, e, i1)
            m2 = jnp.full((_L,), -jnp.inf, jnp.float32)
            i2 = jnp.zeros((_L,), jnp.int32)
            for e in range(_NUM_EXPERTS):
                v = lg_v[e, sl]
                b = jnp.logical_and(i1 != e, v > m2)
                m2 = jnp.where(b, v, m2)
                i2 = jnp.where(b, e, i2)
            p1 = 1.0 / (1.0 + jnp.exp(m2 - m1))
            p2 = 1.0 - p1
            for e in range(_NUM_EXPERTS):
                pt_v[e, sl] = jnp.where(i1 == e, p1,
                                        jnp.where(i2 == e, p2, zero))
            return carry

        lax.fori_loop(0, _CHUNKS, chunk, 0)
        pltpu.sync_copy(pt_v, pt_hbm)


def _row(ref, e, width):
    rows = lax.broadcasted_iota(jnp.int32, (_NUM_EXPERTS, width), 0)
    return jnp.sum(jnp.where(rows == e, ref[...], 0.0), axis=0,
                   keepdims=True)


_SPLIT = 1
_H_SP = _HIDDEN // _SPLIT
_E_SP = _EXPANDED // _SPLIT


def _ffn_all_body(tokens_ref, *refs):
    w1_refs = refs[:_SPLIT]
    b1_ref = refs[_SPLIT]
    w2_refs = refs[_SPLIT + 1:2 * _SPLIT + 1]
    b2_ref = refs[2 * _SPLIT + 1]
    o_ref = refs[2 * _SPLIT + 2]
    e = pl.program_id(0)
    tokens = tokens_ref[...]
    h = _row(b1_ref, e, _EXPANDED)
    for k in range(_SPLIT):
        h += jnp.dot(tokens[:, k * _H_SP:(k + 1) * _H_SP], w1_refs[k][0],
                     preferred_element_type=jnp.float32)
    h = h * 0.5 * (1.0 + lax.erf(h * 0.7071067811865476))
    o = _row(b2_ref, e, _HIDDEN)
    for k in range(_SPLIT):
        o += jnp.dot(h[:, k * _E_SP:(k + 1) * _E_SP], w2_refs[k][0],
                     preferred_element_type=jnp.float32)
    o_ref[0] = o


def _ffn_all(tokens, w1, b1, w2, b2):
    w1_specs = [
        pl.BlockSpec((1, _H_SP, _EXPANDED), lambda e, k=k: (e, k, 0))
        for k in range(_SPLIT)
    ]
    w2_specs = [
        pl.BlockSpec((1, _E_SP, _HIDDEN), lambda e, k=k: (e, k, 0))
        for k in range(_SPLIT)
    ]
    return pl.pallas_call(
        _ffn_all_body,
        grid=(_NUM_EXPERTS,),
        in_specs=[
            pl.BlockSpec((_N_TOKENS, _HIDDEN), lambda e: (0, 0)),
            *w1_specs,
            pl.BlockSpec((_NUM_EXPERTS, _EXPANDED), lambda e: (0, 0)),
            *w2_specs,
            pl.BlockSpec((_NUM_EXPERTS, _HIDDEN), lambda e: (0, 0)),
        ],
        out_specs=pl.BlockSpec((1, _N_TOKENS, _HIDDEN), lambda e: (e, 0, 0)),
        out_shape=jax.ShapeDtypeStruct((_NUM_EXPERTS, _N_TOKENS, _HIDDEN),
                                       tokens.dtype),
    )(tokens, *([w1] * _SPLIT), b1, *([w2] * _SPLIT), b2)


def _combine_body(o_ref, pt_ref, out_ref):
    p = jnp.transpose(pt_ref[...])
    acc = p[:, 0:1] * o_ref[0]
    for e in range(1, _NUM_EXPERTS):
        acc += p[:, e:e + 1] * o_ref[e]
    out_ref[...] = acc


def _combine(o_all, pt):
    return pl.pallas_call(
        _combine_body,
        in_specs=[
            pl.BlockSpec((_NUM_EXPERTS, _N_TOKENS, _HIDDEN),
                         lambda: (0, 0, 0)),
            pl.BlockSpec((_NUM_EXPERTS, _N_TOKENS), lambda: (0, 0)),
        ],
        out_specs=pl.BlockSpec((_N_TOKENS, _HIDDEN), lambda: (0, 0)),
        out_shape=jax.ShapeDtypeStruct((_N_TOKENS, _HIDDEN), o_all.dtype),
    )(o_all, pt)


def kernel(tokens, router_w, router_b, w1, b1, w2, b2):
    logits_t = _compute_logits_t(tokens, router_w, router_b)
    pt = _sc_router(logits_t)
    o_all = _ffn_all(tokens, w1, b1, w2, b2)
    return _combine(o_all, pt)
